# baseline (speedup 1.0000x reference)
import jax
import jax.numpy as jnp
from jax import lax
from jax.experimental import pallas as pl
from jax.experimental.pallas import tpu as pltpu
from jax.experimental.pallas import tpu_sc as plsc

NC, NS, LANES = 2, 16, 16
NW = NC * NS
N0A, N0B = 4096, 4096
B0, B1 = 512, 1024
CW = 128
GROUPS = CW // LANES
HALF = B1 // 2
RCH = 64
NCH = N0A // RCH
PADN = (NCH + 8) * RCH
NBUF = 4
OBS_CH = 64
NOBS = HALF // OBS_CH


def _sc_body(mu0, obs1, idx0, idx1, out_p1, out_b0,
             b0r, b1r, b2r, b3r, i1all, i0all, sortedr, sbins,
             cursor, acc1, bin0, stage, s0, s1, s2, s3):
    c = lax.axis_index("c")
    s = lax.axis_index("s")
    w = c * NS + s
    cb = pl.multiple_of(w * CW, CW)
    iota = lax.iota(jnp.int32, LANES)
    zv = jnp.zeros((LANES,), jnp.float32)
    zi = jnp.zeros((LANES,), jnp.int32)
    ones_i = jnp.ones((LANES,), jnp.int32)
    lane0 = iota == 0
    bufs = (b0r, b1r, b2r, b3r)
    sems = (s0, s1, s2, s3)

    pltpu.sync_copy(idx1, i1all)
    pltpu.sync_copy(idx0, i0all)

    def zcur(i, _):
        cursor[pl.ds(i * LANES, LANES)] = zi
        return 0
    lax.fori_loop(0, (B1 + 2 * LANES) // LANES, zcur, 0)

    def zpad(i, _):
        sortedr[pl.ds(N0A + i * LANES, LANES)] = zi
        sbins[pl.ds(N0A + i * LANES, LANES)] = zi
        return 0
    lax.fori_loop(0, (PADN - N0A) // LANES, zpad, 0)

    def zacc(i, _):
        for g in range(GROUPS):
            acc1[i, pl.ds(g * LANES, LANES)] = zv
        return 0
    lax.fori_loop(0, HALF, zacc, 0)

    def zb0(i, _):
        bin0[pl.ds(i * LANES, LANES)] = zv
        return 0
    lax.fori_loop(0, B0, zb0, 0)

    def hist(gi, _):
        bv = i1all[pl.ds(gi * LANES, LANES)]
        for j in range(LANES):
            plsc.addupdate_scatter(cursor, [jnp.full((LANES,), bv[j], jnp.int32)],
                                   ones_i, mask=lane0)
        return 0
    lax.fori_loop(0, N0A // LANES, hist, 0)

    def pref(gi, run):
        v = cursor[pl.ds(gi * LANES, LANES)]
        cs = plsc.cumsum(v)
        cursor[pl.ds(gi * LANES, LANES)] = cs - v + jnp.full((LANES,), run, jnp.int32)
        return run + cs[LANES - 1]
    lax.fori_loop(0, B1 // LANES, pref, jnp.int32(0))

    nhv = cursor[pl.ds(HALF, LANES)]
    n_half = nhv[0]

    def place(gi, _):
        base = gi * LANES
        bv = i1all[pl.ds(base, LANES)]
        b0v = i0all[pl.ds(base, LANES)]
        for j in range(LANES):
            b = bv[j]
            bfull = jnp.full((LANES,), b, jnp.int32)
            cv = cursor[pl.ds(b, LANES)]
            p = cv[0]
            pfull = jnp.full((LANES,), p, jnp.int32)
            plsc.store_scatter(cursor, [bfull], jnp.full((LANES,), p + 1, jnp.int32), mask=lane0)
            plsc.store_scatter(sortedr, [pfull], jnp.full((LANES,), base + j, jnp.int32), mask=lane0)
            packed = jnp.full((LANES,), b * B0 + b0v[j], jnp.int32)
            plsc.store_scatter(sbins, [pfull], packed, mask=lane0)
        return 0
    lax.fori_loop(0, N0A // LANES, place, 0)

    def g_src(k):
        return mu0.at[sortedr.at[pl.ds(k * RCH, RCH)], pl.ds(cb, CW)]

    def make_consume(buf):
        def consume2(k, phase_b):
            def grp(j16, _):
                p16 = k * RCH + j16 * LANES
                pkv = sbins[pl.ds(p16, LANES)]
                for j in range(LANES):
                    p = p16 + j
                    pv = jnp.full((LANES,), p, jnp.int32)
                    if phase_b:
                        msk = (pv >= jnp.full((LANES,), n_half, jnp.int32)) & (pv < N0A)
                    else:
                        msk = pv < jnp.full((LANES,), n_half, jnp.int32)
                    pk = pkv[j]
                    row1 = jnp.full((LANES,), (pk >> 9) & (HALF - 1), jnp.int32)
                    rsv = zv
                    for g in range(GROUPS):
                        xv = buf[j16 * LANES + j, pl.ds(g * LANES, LANES)]
                        plsc.addupdate_scatter(acc1, [row1, iota + g * LANES], xv, mask=msk)
                        rsv = rsv + xv
                    row0 = jnp.full((LANES,), (pk & (B0 - 1)) * LANES, jnp.int32)
                    plsc.addupdate_scatter(bin0, [row0 + iota], rsv, mask=msk)
                return 0
            lax.fori_loop(0, RCH // LANES, grp, 0)
        return consume2

    consumers = tuple(make_consume(b) for b in bufs)

    def run_phase(first_chunk, ngroups, phase_b):
        for b in range(NBUF):
            pltpu.async_copy(g_src(first_chunk + b), bufs[b], sems[b])

        def ring(g, _):
            for b in range(NBUF):
                k = first_chunk + g * NBUF + b
                pltpu.make_async_copy(g_src(k), bufs[b], sems[b]).wait()
                consumers[b](k, phase_b)
                pltpu.async_copy(g_src(k + NBUF), bufs[b], sems[b])
            return 0
        lax.fori_loop(0, ngroups, ring, 0)
        for b in range(NBUF):
            k = first_chunk + ngroups * NBUF + b
            pltpu.make_async_copy(g_src(k), bufs[b], sems[b]).wait()

    def sse_drain(bin_base, carry):
        def o_src(kb):
            return obs1.at[pl.ds(bin_base + kb * OBS_CH, OBS_CH), pl.ds(cb, CW)]
        for b in range(NBUF):
            pltpu.async_copy(o_src(b), bufs[b], sems[b])
        for grp4 in range(NOBS // NBUF):
            for b in range(NBUF):
                kb = grp4 * NBUF + b
                pltpu.make_async_copy(o_src(kb), bufs[b], sems[b]).wait()

                def sse_row(rb, acc, kb=kb, b=b):
                    for g in range(GROUPS):
                        a = acc1[kb * OBS_CH + rb, pl.ds(g * LANES, LANES)]
                        o = bufs[b][rb, pl.ds(g * LANES, LANES)]
                        d = o - a
                        acc = acc + d * d
                    return acc
                carry = lax.fori_loop(0, OBS_CH, sse_row, carry)
                if grp4 + 1 < NOBS // NBUF:
                    pltpu.async_copy(o_src(kb + NBUF), bufs[b], sems[b])
        return carry

    gA = (((n_half + RCH - 1) // RCH) + NBUF - 1) // NBUF
    run_phase(0, gA, False)
    loss = sse_drain(0, zv)
    lax.fori_loop(0, HALF, zacc, 0)

    fB = n_half // RCH
    gB = ((NCH - fB) + NBUF - 1) // NBUF
    run_phase(fB, gB, True)
    loss = sse_drain(HALF, loss)

    for g in range(GROUPS):
        stage[pl.ds(g * LANES, LANES)] = zv
    stage[pl.ds(0, LANES)] = loss
    pltpu.sync_copy(stage, out_p1.at[pl.ds(w * CW, CW)])
    pltpu.sync_copy(bin0, out_b0.at[pl.ds(w * B0 * LANES, B0 * LANES)])


def _sc_call(mu0, obs1, idx0, idx1):
    mesh = plsc.VectorSubcoreMesh(core_axis_name="c", subcore_axis_name="s",
                                  num_cores=NC, num_subcores=NS)
    return pl.kernel(
        _sc_body,
        out_type=(jax.ShapeDtypeStruct((NW * CW,), jnp.float32),
                  jax.ShapeDtypeStruct((NW * B0 * LANES,), jnp.float32)),
        mesh=mesh,
        compiler_params=pltpu.CompilerParams(needs_layout_passes=False),
        scratch_types=[
            pltpu.VMEM((RCH, CW), jnp.float32),
            pltpu.VMEM((RCH, CW), jnp.float32),
            pltpu.VMEM((RCH, CW), jnp.float32),
            pltpu.VMEM((RCH, CW), jnp.float32),
            pltpu.VMEM((N0A,), jnp.int32),
            pltpu.VMEM((N0A,), jnp.int32),
            pltpu.VMEM((PADN,), jnp.int32),
            pltpu.VMEM((PADN,), jnp.int32),
            pltpu.VMEM((B1 + 2 * LANES,), jnp.int32),
            pltpu.VMEM((HALF, CW), jnp.float32),
            pltpu.VMEM((B0 * LANES,), jnp.float32),
            pltpu.VMEM((CW,), jnp.float32),
            pltpu.SemaphoreType.DMA,
            pltpu.SemaphoreType.DMA,
            pltpu.SemaphoreType.DMA,
            pltpu.SemaphoreType.DMA,
        ],
    )(mu0, obs1, idx0, idx1)


def _tc_body(mu1_ref, map2_ref, obs0_ref, obs2_ref, p1_ref, b0p_ref, out_ref):
    colsum = jnp.sum(mu1_ref[...], axis=0, keepdims=True)
    proj = lax.dot_general(map2_ref[...], colsum,
                           (((1,), (1,)), ((), ())),
                           preferred_element_type=jnp.float32)
    d2 = obs2_ref[...] - proj[:, 0]
    sse2 = jnp.sum(d2 * d2)
    bin0 = jnp.sum(b0p_ref[...], axis=(0, 2))
    d0 = obs0_ref[...] - bin0
    sse0 = jnp.sum(d0 * d0)
    sse1 = jnp.sum(p1_ref[...])
    total = 0.5 * (sse0 + sse1) + sse2
    out_ref[...] = jnp.reshape(total, (1, 1))


def _tc_call(mu1, mapping2, obs0, obs2, p1, b0p):
    return pl.pallas_call(
        _tc_body,
        out_shape=jax.ShapeDtypeStruct((1, 1), jnp.float32),
    )(mu1, mapping2, obs0, obs2, p1, b0p)


def kernel(mu0, mu1, obs0, obs1, obs2, idx0, idx1, mapping2):
    p1, b0p = _sc_call(mu0, obs1, idx0, idx1)
    loss = _tc_call(mu1, mapping2, obs0, obs2,
                    p1.reshape(NW, CW), b0p.reshape(NW, B0, LANES))
    return loss[0, 0]


# speedup vs baseline: 1.4685x; 1.4685x over previous
import jax
import jax.numpy as jnp
from jax import lax
from jax.experimental import pallas as pl
from jax.experimental.pallas import tpu as pltpu
from jax.experimental.pallas import tpu_sc as plsc

NC, NS, LANES = 2, 16, 16
NW = NC * NS
N0A, N0B = 4096, 4096
B0, B1 = 512, 1024
CW = 64
NSWEEP = N0B // (NW * CW)
RCHUNK = 128
NCHUNK = N0A // RCHUNK
GROUPS = CW // LANES
SSE_BCHUNK = 128
NSSE = B1 // SSE_BCHUNK
NBUF = 4


def _sc_body(mu0, obs1, idx0, idx1, out_p1, out_b0,
             b0, b1, b2, b3, i1all, i0all, acc1, bin0, stage,
             s0, s1, s2, s3):
    c = lax.axis_index("c")
    s = lax.axis_index("s")
    w = c * NS + s
    iota = lax.iota(jnp.int32, LANES)
    zv = jnp.zeros((LANES,), jnp.float32)
    bufs = (b0, b1, b2, b3)
    sems = (s0, s1, s2, s3)

    pltpu.sync_copy(idx1, i1all)
    pltpu.sync_copy(idx0, i0all)

    def z1(i, _):
        for g in range(GROUPS):
            acc1[i, pl.ds(g * LANES, LANES)] = zv
        return 0
    lax.fori_loop(0, B1, z1, 0)

    def z0(i, _):
        bin0[i, :] = zv
        return 0
    lax.fori_loop(0, B0, z0, 0)

    loss = zv

    for sweep in range(NSWEEP):
        cbase = sweep * (NW * CW) + w * CW

        def mu_src(k):
            return mu0.at[pl.ds(k * RCHUNK, RCHUNK), pl.ds(cbase, CW)]

        def scatter_chunk(k, buf):
            def grp(j16, _):
                r0 = k * RCHUNK + j16 * LANES
                bv = i1all[pl.ds(r0, LANES)]
                b0v = i0all[pl.ds(r0, LANES)]
                for j in range(LANES):
                    r = j16 * LANES + j
                    row1 = jnp.full((LANES,), bv[j], jnp.int32)
                    g0 = buf[r, pl.ds(0, LANES)]
                    g1 = buf[r, pl.ds(LANES, LANES)]
                    g2 = buf[r, pl.ds(2 * LANES, LANES)]
                    g3 = buf[r, pl.ds(3 * LANES, LANES)]
                    plsc.addupdate_scatter(acc1, [row1, iota], g0)
                    plsc.addupdate_scatter(acc1, [row1, iota + LANES], g1)
                    plsc.addupdate_scatter(acc1, [row1, iota + 2 * LANES], g2)
                    plsc.addupdate_scatter(acc1, [row1, iota + 3 * LANES], g3)
                    rsv = (g0 + g1) + (g2 + g3)
                    row0 = jnp.full((LANES,), b0v[j], jnp.int32)
                    plsc.addupdate_scatter(bin0, [row0, iota], rsv)
                return 0
            lax.fori_loop(0, RCHUNK // LANES, grp, 0)

        for b in range(NBUF):
            pltpu.async_copy(mu_src(b), bufs[b], sems[b])

        def ring_iter(nb, _):
            for b in range(NBUF):
                k = nb * NBUF + b
                pltpu.make_async_copy(mu_src(k), bufs[b], sems[b]).wait()
                scatter_chunk(k, bufs[b])
                pltpu.async_copy(mu_src(k + NBUF), bufs[b], sems[b])
            return 0
        lax.fori_loop(0, NCHUNK // NBUF - 1, ring_iter, 0)
        for b in range(NBUF):
            k = NCHUNK - NBUF + b
            pltpu.make_async_copy(mu_src(k), bufs[b], sems[b]).wait()
            scatter_chunk(k, bufs[b])

        def obs_src(kb):
            return obs1.at[pl.ds(kb * SSE_BCHUNK, SSE_BCHUNK), pl.ds(cbase, CW)]

        def sse_chunk(kb, buf, carry):
            def sse_row(rb, acc):
                for g in range(GROUPS):
                    a = acc1[kb * SSE_BCHUNK + rb, pl.ds(g * LANES, LANES)]
                    o = buf[rb, pl.ds(g * LANES, LANES)]
                    d = o - a
                    acc = acc + d * d
                return acc
            return lax.fori_loop(0, SSE_BCHUNK, sse_row, carry)

        for b in range(NBUF):
            pltpu.async_copy(obs_src(b), bufs[b], sems[b])

        def sse_ring(nb, carry):
            for b in range(NBUF):
                kb = nb * NBUF + b
                pltpu.make_async_copy(obs_src(kb), bufs[b], sems[b]).wait()
                carry = sse_chunk(kb, bufs[b], carry)
                pltpu.async_copy(obs_src(kb + NBUF), bufs[b], sems[b])
            return carry
        loss = lax.fori_loop(0, NSSE // NBUF - 1, sse_ring, loss)
        for b in range(NBUF):
            kb = NSSE - NBUF + b
            pltpu.make_async_copy(obs_src(kb), bufs[b], sems[b]).wait()
            loss = sse_chunk(kb, bufs[b], loss)

        if sweep + 1 < NSWEEP:
            lax.fori_loop(0, B1, z1, 0)

    stage[:] = loss
    pltpu.sync_copy(stage, out_p1.at[w])
    pltpu.sync_copy(bin0, out_b0.at[w])


def _sc_call(mu0, obs1, idx0, idx1):
    mesh = plsc.VectorSubcoreMesh(core_axis_name="c", subcore_axis_name="s",
                                  num_cores=NC, num_subcores=NS)
    return pl.kernel(
        _sc_body,
        out_type=(jax.ShapeDtypeStruct((NW, LANES), jnp.float32),
                  jax.ShapeDtypeStruct((NW, B0, LANES), jnp.float32)),
        mesh=mesh,
        compiler_params=pltpu.CompilerParams(needs_layout_passes=False,
                                             use_tc_tiling_on_sc=False),
        scratch_types=[
            pltpu.VMEM((RCHUNK, CW), jnp.float32),
            pltpu.VMEM((RCHUNK, CW), jnp.float32),
            pltpu.VMEM((RCHUNK, CW), jnp.float32),
            pltpu.VMEM((RCHUNK, CW), jnp.float32),
            pltpu.VMEM((N0A,), jnp.int32),
            pltpu.VMEM((N0A,), jnp.int32),
            pltpu.VMEM((B1, CW), jnp.float32),
            pltpu.VMEM((B0, LANES), jnp.float32),
            pltpu.VMEM((LANES,), jnp.float32),
            pltpu.SemaphoreType.DMA,
            pltpu.SemaphoreType.DMA,
            pltpu.SemaphoreType.DMA,
            pltpu.SemaphoreType.DMA,
        ],
    )(mu0, obs1, idx0, idx1)


def _tc_pre_body(mu1_ref, map2_ref, obs2_ref, out_ref):
    colsum = jnp.sum(mu1_ref[...], axis=0, keepdims=True)
    proj = lax.dot_general(map2_ref[...], colsum,
                           (((1,), (1,)), ((), ())),
                           preferred_element_type=jnp.float32)
    d2 = obs2_ref[...] - proj[:, 0]
    sse2 = jnp.sum(d2 * d2)
    out_ref[...] = jnp.reshape(sse2, (1, 1))


def _tc_post_body(obs0_ref, p1_ref, b0p_ref, sse2_ref, out_ref):
    bin0 = jnp.sum(b0p_ref[...], axis=(0, 2))
    d0 = obs0_ref[...] - bin0
    sse0 = jnp.sum(d0 * d0)
    sse1 = jnp.sum(p1_ref[...])
    total = 0.5 * (sse0 + sse1) + sse2_ref[0, 0]
    out_ref[...] = jnp.reshape(total, (1, 1))


def kernel(mu0, mu1, obs0, obs1, obs2, idx0, idx1, mapping2):
    sse2 = pl.pallas_call(
        _tc_pre_body,
        out_shape=jax.ShapeDtypeStruct((1, 1), jnp.float32),
    )(mu1, mapping2, obs2)
    p1, b0p = _sc_call(mu0, obs1, idx0, idx1)
    loss = pl.pallas_call(
        _tc_post_body,
        out_shape=jax.ShapeDtypeStruct((1, 1), jnp.float32),
    )(obs0, p1, b0p, sse2)
    return loss[0, 0]


# speedup vs baseline: 1.4810x; 1.0085x over previous
import jax
import jax.numpy as jnp
from jax import lax
from jax.experimental import pallas as pl
from jax.experimental.pallas import tpu as pltpu
from jax.experimental.pallas import tpu_sc as plsc

NC, NS, LANES = 2, 16, 16
NW = NC * NS
N0A, N0B = 4096, 4096
B0, B1 = 512, 1024
CW = 64
NSWEEP = N0B // (NW * CW)
RCHUNK = 128
NCHUNK = N0A // RCHUNK
GROUPS = CW // LANES
SSE_BCHUNK = 128
NSSE = B1 // SSE_BCHUNK
NBUF = 4


def _sc_body(mu0, obs1, idx0, idx1, out_p1, out_b0,
             b0, b1, b2, b3, i1all, i0all, acc1, bin0, stage,
             s0, s1, s2, s3):
    c = lax.axis_index("c")
    s = lax.axis_index("s")
    w = c * NS + s
    iota = lax.iota(jnp.int32, LANES)
    zv = jnp.zeros((LANES,), jnp.float32)
    bufs = (b0, b1, b2, b3)
    sems = (s0, s1, s2, s3)

    cbase0 = w * CW
    for _b in range(NBUF):
        pltpu.async_copy(
            mu0.at[pl.ds(_b * RCHUNK, RCHUNK), pl.ds(cbase0, CW)], bufs[_b], sems[_b])

    pltpu.sync_copy(idx1, i1all)
    pltpu.sync_copy(idx0, i0all)

    def z1(i, _):
        for g in range(GROUPS):
            acc1[i, pl.ds(g * LANES, LANES)] = zv
        return 0
    lax.fori_loop(0, B1, z1, 0)

    def z0(i, _):
        bin0[i, :] = zv
        return 0
    lax.fori_loop(0, B0, z0, 0)

    loss = zv

    for sweep in range(NSWEEP):
        cbase = sweep * (NW * CW) + w * CW

        def mu_src(k):
            return mu0.at[pl.ds(k * RCHUNK, RCHUNK), pl.ds(cbase, CW)]

        def scatter_chunk(k, buf):
            def grp(j16, _):
                r0 = k * RCHUNK + j16 * LANES
                bv = i1all[pl.ds(r0, LANES)]
                b0v = i0all[pl.ds(r0, LANES)]
                for j in range(LANES):
                    r = j16 * LANES + j
                    row1 = jnp.full((LANES,), bv[j], jnp.int32)
                    g0 = buf[r, pl.ds(0, LANES)]
                    g1 = buf[r, pl.ds(LANES, LANES)]
                    g2 = buf[r, pl.ds(2 * LANES, LANES)]
                    g3 = buf[r, pl.ds(3 * LANES, LANES)]
                    plsc.addupdate_scatter(acc1, [row1, iota], g0)
                    plsc.addupdate_scatter(acc1, [row1, iota + LANES], g1)
                    plsc.addupdate_scatter(acc1, [row1, iota + 2 * LANES], g2)
                    plsc.addupdate_scatter(acc1, [row1, iota + 3 * LANES], g3)
                    rsv = (g0 + g1) + (g2 + g3)
                    row0 = jnp.full((LANES,), b0v[j], jnp.int32)
                    plsc.addupdate_scatter(bin0, [row0, iota], rsv)
                return 0
            lax.fori_loop(0, RCHUNK // LANES, grp, 0)

        if sweep > 0:
            for b in range(NBUF):
                pltpu.async_copy(mu_src(b), bufs[b], sems[b])

        def ring_iter(nb, _):
            for b in range(NBUF):
                k = nb * NBUF + b
                pltpu.make_async_copy(mu_src(k), bufs[b], sems[b]).wait()
                scatter_chunk(k, bufs[b])
                pltpu.async_copy(mu_src(k + NBUF), bufs[b], sems[b])
            return 0
        lax.fori_loop(0, NCHUNK // NBUF - 1, ring_iter, 0)
        def obs_src(kb):
            return obs1.at[pl.ds(kb * SSE_BCHUNK, SSE_BCHUNK), pl.ds(cbase, CW)]

        for b in range(NBUF):
            k = NCHUNK - NBUF + b
            pltpu.make_async_copy(mu_src(k), bufs[b], sems[b]).wait()
            scatter_chunk(k, bufs[b])
            pltpu.async_copy(obs_src(b), bufs[b], sems[b])


        def sse_chunk(kb, buf, carry):
            def sse_row(rb, acc):
                for g in range(GROUPS):
                    a = acc1[kb * SSE_BCHUNK + rb, pl.ds(g * LANES, LANES)]
                    o = buf[rb, pl.ds(g * LANES, LANES)]
                    d = o - a
                    acc = acc + d * d
                return acc
            return lax.fori_loop(0, SSE_BCHUNK, sse_row, carry)

        def sse_ring(nb, carry):
            for b in range(NBUF):
                kb = nb * NBUF + b
                pltpu.make_async_copy(obs_src(kb), bufs[b], sems[b]).wait()
                carry = sse_chunk(kb, bufs[b], carry)
                pltpu.async_copy(obs_src(kb + NBUF), bufs[b], sems[b])
            return carry
        loss = lax.fori_loop(0, NSSE // NBUF - 1, sse_ring, loss)
        for b in range(NBUF):
            kb = NSSE - NBUF + b
            pltpu.make_async_copy(obs_src(kb), bufs[b], sems[b]).wait()
            loss = sse_chunk(kb, bufs[b], loss)

        if sweep + 1 < NSWEEP:
            lax.fori_loop(0, B1, z1, 0)

    stage[:] = loss
    pltpu.sync_copy(stage, out_p1.at[w])
    pltpu.sync_copy(bin0, out_b0.at[w])


def _sc_call(mu0, obs1, idx0, idx1):
    mesh = plsc.VectorSubcoreMesh(core_axis_name="c", subcore_axis_name="s",
                                  num_cores=NC, num_subcores=NS)
    return pl.kernel(
        _sc_body,
        out_type=(jax.ShapeDtypeStruct((NW, LANES), jnp.float32),
                  jax.ShapeDtypeStruct((NW, B0, LANES), jnp.float32)),
        mesh=mesh,
        compiler_params=pltpu.CompilerParams(needs_layout_passes=False,
                                             use_tc_tiling_on_sc=False),
        scratch_types=[
            pltpu.VMEM((RCHUNK, CW), jnp.float32),
            pltpu.VMEM((RCHUNK, CW), jnp.float32),
            pltpu.VMEM((RCHUNK, CW), jnp.float32),
            pltpu.VMEM((RCHUNK, CW), jnp.float32),
            pltpu.VMEM((N0A,), jnp.int32),
            pltpu.VMEM((N0A,), jnp.int32),
            pltpu.VMEM((B1, CW), jnp.float32),
            pltpu.VMEM((B0, LANES), jnp.float32),
            pltpu.VMEM((LANES,), jnp.float32),
            pltpu.SemaphoreType.DMA,
            pltpu.SemaphoreType.DMA,
            pltpu.SemaphoreType.DMA,
            pltpu.SemaphoreType.DMA,
        ],
    )(mu0, obs1, idx0, idx1)


def _tc_pre_body(mu1_ref, map2_ref, obs2_ref, out_ref):
    colsum = jnp.sum(mu1_ref[...], axis=0, keepdims=True)
    proj = lax.dot_general(map2_ref[...], colsum,
                           (((1,), (1,)), ((), ())),
                           preferred_element_type=jnp.float32)
    d2 = obs2_ref[...] - proj[:, 0]
    sse2 = jnp.sum(d2 * d2)
    out_ref[...] = jnp.reshape(sse2, (1, 1))


def _tc_post_body(obs0_ref, p1_ref, b0p_ref, sse2_ref, out_ref):
    bin0 = jnp.sum(b0p_ref[...], axis=(0, 2))
    d0 = obs0_ref[...] - bin0
    sse0 = jnp.sum(d0 * d0)
    sse1 = jnp.sum(p1_ref[...])
    total = 0.5 * (sse0 + sse1) + sse2_ref[0, 0]
    out_ref[...] = jnp.reshape(total, (1, 1))


def kernel(mu0, mu1, obs0, obs1, obs2, idx0, idx1, mapping2):
    sse2 = pl.pallas_call(
        _tc_pre_body,
        out_shape=jax.ShapeDtypeStruct((1, 1), jnp.float32),
    )(mu1, mapping2, obs2)
    p1, b0p = _sc_call(mu0, obs1, idx0, idx1)
    loss = pl.pallas_call(
        _tc_post_body,
        out_shape=jax.ShapeDtypeStruct((1, 1), jnp.float32),
    )(obs0, p1, b0p, sse2)
    return loss[0, 0]
